# baseline (speedup 1.0000x reference)
import functools

import jax
import jax.numpy as jnp
from jax import lax
from jax.experimental import pallas as pl
from jax.experimental.pallas import tpu as pltpu
from jax.experimental.pallas import tpu_sc as plsc


@functools.lru_cache(maxsize=None)
def _build_sc_kernel(S, T, B, D):
    info = plsc.get_sparse_core_info()
    NC, NS, L = info.num_cores, info.num_subcores, info.num_lanes
    NW = NC * NS
    PQ = L // D
    CW = 128
    BR = CW // D
    NCB = (B * D) // CW
    NSH = NW // NCB
    SQ = S // NSH
    NG = SQ // L
    NP = SQ // PQ
    KR = CW // L

    mesh = plsc.VectorSubcoreMesh(core_axis_name="c", subcore_axis_name="s")

    @functools.partial(
        pl.kernel,
        mesh=mesh,
        out_type=jax.ShapeDtypeStruct((B, S * D), jnp.float32),
        compiler_params=pltpu.CompilerParams(needs_layout_passes=False),
        scratch_types=[
            pltpu.VMEM((T, CW), jnp.float32),
            pltpu.VMEM((T, CW), jnp.float32),
            pltpu.VMEM((SQ,), jnp.float32),
            pltpu.VMEM((SQ,), jnp.int32),
            pltpu.VMEM((SQ,), jnp.float32),
            pltpu.VMEM((BR, SQ * D), jnp.float32),
            pltpu.SemaphoreType.DMA,
            pltpu.SemaphoreType.DMA,
            pltpu.SemaphoreType.DMA,
        ],
    )
    def sc_kernel(q_hbm, knots_hbm, x0_hbm, x1_hbm, sd_hbm, out_hbm,
                  xt2, sdb, q_v, pos_v, w_v, outf,
                  sem_q, sem_sd, sem_xt):
        wid = lax.axis_index("s") * NC + lax.axis_index("c")
        cb = lax.rem(wid, NCB)
        sh = lax.div(wid, NCB)
        col0 = cb * CW
        s0 = sh * SQ

        cp_q = pltpu.async_copy(q_hbm.at[pl.ds(s0, SQ)], q_v, sem_q)
        cp_sd = pltpu.async_copy(sd_hbm.at[:, pl.ds(0, CW)], sdb, sem_sd)
        cp_x0 = pltpu.async_copy(x0_hbm.at[:, pl.ds(col0, CW)],
                                 xt2.at[pl.ds(0, 1), :], sem_xt)
        cp_kn = pltpu.async_copy(knots_hbm.at[:, pl.ds(col0, CW)],
                                 xt2.at[pl.ds(1, T - 2), :], sem_xt)
        cp_x1 = pltpu.async_copy(x1_hbm.at[:, pl.ds(col0, CW)],
                                 xt2.at[pl.ds(T - 1, 1), :], sem_xt)
        cp_q.wait()
        cp_sd.wait()

        lane = lax.iota(jnp.int32, L)
        ones = jnp.ones((L,), jnp.int32)
        zero = jnp.zeros((L,), jnp.int32)

        ts = [sdb[t, pl.ds(0, L)] for t in range(T)]
        spacing = ts[1] - ts[0] + 1e-10

        for g in range(NG):
            q = q_v[pl.ds(g * L, L)]
            acc = zero
            tl = ts[0]
            for t in range(1, T):
                c = ts[t] < q
                acc = acc + jnp.where(c, ones, zero)
                tl = jnp.where(c, ts[t], tl)
            pos_v[pl.ds(g * L, L)] = jnp.minimum(acc, T - 2)
            w_v[pl.ds(g * L, L)] = (q - tl) / spacing

        cp_x0.wait()
        cp_kn.wait()
        cp_x1.wait()

        svec = lax.shift_right_logical(lane, 3)
        dvec = lane & (D - 1)

        @plsc.parallel_loop(0, NG, 1, unroll=2)
        def interp(g):
            posg = pos_v[pl.ds(g * L, L)]
            wg = w_v[pl.ds(g * L, L)]
            for j in range(L):
                l = posg[j]
                wq = wg[j]
                inner = jnp.full((L,), (g * L + j) * D, jnp.int32) + dvec
                for k in range(KR):
                    a = xt2[l, pl.ds(k * L, L)]
                    b = xt2[l + 1, pl.ds(k * L, L)]
                    y = a + wq * (b - a)
                    plsc.store_scatter(outf, [svec + 2 * k, inner], y)

        pltpu.sync_copy(outf,
                        out_hbm.at[pl.ds(cb * BR, BR),
                                   pl.ds(s0 * D, SQ * D)])

    return sc_kernel


def kernel(query_t, knots, x0, x1, spline_discr):
    (S,) = query_t.shape
    TK, B, D = knots.shape
    T = TK + 2
    sck = _build_sc_kernel(S, T, B, D)
    out2 = sck(
        query_t,
        knots.reshape(TK, B * D),
        x0.reshape(1, B * D),
        x1.reshape(1, B * D),
        spline_discr,
    )
    return out2.reshape(B, S, D)


# speedup vs baseline: 1.3437x; 1.3437x over previous
import functools

import jax
import jax.numpy as jnp
from jax import lax
from jax.experimental import pallas as pl
from jax.experimental.pallas import tpu as pltpu
from jax.experimental.pallas import tpu_sc as plsc


@functools.lru_cache(maxsize=None)
def _build_sc_kernel(S, T, B, D):
    info = plsc.get_sparse_core_info()
    NC, NS, L = info.num_cores, info.num_subcores, info.num_lanes
    NW = NC * NS
    PQ = L // D
    CW = 128
    BR = CW // D
    NCB = (B * D) // CW
    NSH = NW // NCB
    SQ = S // NSH
    NG = SQ // L
    NP = SQ // PQ
    KR = CW // L

    mesh = plsc.VectorSubcoreMesh(core_axis_name="c", subcore_axis_name="s")

    @functools.partial(
        pl.kernel,
        mesh=mesh,
        out_type=jax.ShapeDtypeStruct((B, S * D), jnp.float32),
        compiler_params=pltpu.CompilerParams(needs_layout_passes=False),
        scratch_types=[
            pltpu.VMEM((T, CW), jnp.float32),
            pltpu.VMEM((T, CW), jnp.float32),
            pltpu.VMEM((SQ,), jnp.float32),
            pltpu.VMEM((SQ,), jnp.int32),
            pltpu.VMEM((SQ,), jnp.float32),
            pltpu.VMEM((BR, SQ * D), jnp.float32),
            pltpu.SemaphoreType.DMA,
            pltpu.SemaphoreType.DMA,
            pltpu.SemaphoreType.DMA,
        ],
    )
    def sc_kernel(q_hbm, knots_hbm, x0_hbm, x1_hbm, sd_hbm, out_hbm,
                  xt2, sdb, q_v, pos_v, w_v, outf,
                  sem_q, sem_sd, sem_xt):
        wid = lax.axis_index("s") * NC + lax.axis_index("c")
        cb = lax.rem(wid, NCB)
        sh = lax.div(wid, NCB)
        col0 = cb * CW
        s0 = sh * SQ

        cp_q = pltpu.async_copy(q_hbm.at[pl.ds(s0, SQ)], q_v, sem_q)
        cp_sd = pltpu.async_copy(sd_hbm.at[:, pl.ds(0, CW)], sdb, sem_sd)
        cp_x0 = pltpu.async_copy(x0_hbm.at[:, pl.ds(col0, CW)],
                                 xt2.at[pl.ds(0, 1), :], sem_xt)
        cp_kn = pltpu.async_copy(knots_hbm.at[:, pl.ds(col0, CW)],
                                 xt2.at[pl.ds(1, T - 2), :], sem_xt)
        cp_x1 = pltpu.async_copy(x1_hbm.at[:, pl.ds(col0, CW)],
                                 xt2.at[pl.ds(T - 1, 1), :], sem_xt)
        cp_q.wait()
        cp_sd.wait()

        lane = lax.iota(jnp.int32, L)
        ones = jnp.ones((L,), jnp.int32)
        zero = jnp.zeros((L,), jnp.int32)

        ts = [sdb[t, pl.ds(0, L)] for t in range(T)]
        spacing = ts[1] - ts[0] + 1e-10

        for g in range(NG):
            q = q_v[pl.ds(g * L, L)]
            acc = zero
            tl = ts[0]
            for t in range(1, T):
                c = ts[t] < q
                acc = acc + jnp.where(c, ones, zero)
                tl = jnp.where(c, ts[t], tl)
            pos_v[pl.ds(g * L, L)] = jnp.minimum(acc, T - 2)
            w_v[pl.ds(g * L, L)] = (q - tl) / spacing

        cp_x0.wait()
        cp_kn.wait()
        cp_x1.wait()

        svec = lax.shift_right_logical(lane, 3)
        dvec = lane & (D - 1)

        @plsc.parallel_loop(0, NP, 1, unroll=4)
        def interp(i):
            sv = svec + i * PQ
            pos2 = plsc.load_gather(pos_v, [sv])
            w2 = plsc.load_gather(w_v, [sv])
            pos2b = pos2 + 1
            inner = sv * D + dvec
            for b in range(BR):
                bspl = jnp.full((L,), b, jnp.int32)
                cvec = dvec + b * D
                a = plsc.load_gather(xt2, [pos2, cvec])
                bb = plsc.load_gather(xt2, [pos2b, cvec])
                y = a + w2 * (bb - a)
                plsc.store_scatter(outf, [bspl, inner], y)

        pltpu.sync_copy(outf,
                        out_hbm.at[pl.ds(cb * BR, BR),
                                   pl.ds(s0 * D, SQ * D)])

    return sc_kernel


def kernel(query_t, knots, x0, x1, spline_discr):
    (S,) = query_t.shape
    TK, B, D = knots.shape
    T = TK + 2
    sck = _build_sc_kernel(S, T, B, D)
    out2 = sck(
        query_t,
        knots.reshape(TK, B * D),
        x0.reshape(1, B * D),
        x1.reshape(1, B * D),
        spline_discr,
    )
    return out2.reshape(B, S, D)


# speedup vs baseline: 1.3902x; 1.0346x over previous
import functools

import jax
import jax.numpy as jnp
from jax import lax
from jax.experimental import pallas as pl
from jax.experimental.pallas import tpu as pltpu
from jax.experimental.pallas import tpu_sc as plsc


@functools.lru_cache(maxsize=None)
def _build_sc_kernel(S, T, B, D):
    info = plsc.get_sparse_core_info()
    NC, NS, L = info.num_cores, info.num_subcores, info.num_lanes
    NW = NC * NS
    PQ = L // D
    CW = 128
    BR = CW // D
    NCB = (B * D) // CW
    NSH = NW // NCB
    SQ = S // NSH
    NG = SQ // L
    NP = SQ // PQ
    KR = CW // L

    mesh = plsc.VectorSubcoreMesh(core_axis_name="c", subcore_axis_name="s")

    @functools.partial(
        pl.kernel,
        mesh=mesh,
        out_type=jax.ShapeDtypeStruct((B, S * D), jnp.float32),
        compiler_params=pltpu.CompilerParams(needs_layout_passes=False),
        scratch_types=[
            pltpu.VMEM((T, CW), jnp.float32),
            pltpu.VMEM((T, CW), jnp.float32),
            pltpu.VMEM((SQ,), jnp.float32),
            pltpu.VMEM((SQ,), jnp.int32),
            pltpu.VMEM((SQ,), jnp.float32),
            pltpu.VMEM((BR, SQ * D), jnp.float32),
            pltpu.SemaphoreType.DMA,
            pltpu.SemaphoreType.DMA,
            pltpu.SemaphoreType.DMA,
        ],
    )
    def sc_kernel(q_hbm, xt_hbm, sd_hbm, out_hbm,
                  xt2, sdb, q_v, pos_v, w_v, outf,
                  sem_q, sem_sd, sem_xt):
        wid = lax.axis_index("s") * NC + lax.axis_index("c")
        cb = lax.rem(wid, NCB)
        sh = lax.div(wid, NCB)
        col0 = cb * CW
        s0 = sh * SQ

        cp_q = pltpu.async_copy(q_hbm.at[pl.ds(s0, SQ)], q_v, sem_q)
        cp_sd = pltpu.async_copy(sd_hbm.at[:, pl.ds(0, CW)], sdb, sem_sd)
        cp_xt = pltpu.async_copy(xt_hbm.at[:, pl.ds(col0, CW)], xt2, sem_xt)
        cp_q.wait()
        cp_sd.wait()

        lane = lax.iota(jnp.int32, L)
        ones = jnp.ones((L,), jnp.int32)
        zero = jnp.zeros((L,), jnp.int32)

        ts = [sdb[t, pl.ds(0, L)] for t in range(T)]
        spacing = ts[1] - ts[0] + 1e-10

        for g in range(NG):
            q = q_v[pl.ds(g * L, L)]
            acc = zero
            tl = ts[0]
            for t in range(1, T):
                c = ts[t] < q
                acc = acc + jnp.where(c, ones, zero)
                tl = jnp.where(c, ts[t], tl)
            pos_v[pl.ds(g * L, L)] = jnp.minimum(acc, T - 2)
            w_v[pl.ds(g * L, L)] = (q - tl) / spacing

        cp_xt.wait()

        svec = lax.shift_right_logical(lane, 3)
        dvec = lane & (D - 1)

        @plsc.parallel_loop(0, NP, 1, unroll=4)
        def interp(i):
            sv = svec + i * PQ
            pos2 = plsc.load_gather(pos_v, [sv])
            w2 = plsc.load_gather(w_v, [sv])
            pos2b = pos2 + 1
            inner = sv * D + dvec
            for b in range(BR):
                bspl = jnp.full((L,), b, jnp.int32)
                cvec = dvec + b * D
                a = plsc.load_gather(xt2, [pos2, cvec])
                bb = plsc.load_gather(xt2, [pos2b, cvec])
                y = a + w2 * (bb - a)
                plsc.store_scatter(outf, [bspl, inner], y)

        pltpu.sync_copy(outf,
                        out_hbm.at[pl.ds(cb * BR, BR),
                                   pl.ds(s0 * D, SQ * D)])

    return sc_kernel


def kernel(query_t, knots, x0, x1, spline_discr):
    (S,) = query_t.shape
    TK, B, D = knots.shape
    T = TK + 2
    sck = _build_sc_kernel(S, T, B, D)
    xt = jnp.concatenate([x0, knots, x1], axis=0).reshape(T, B * D)
    out2 = sck(query_t, xt, spline_discr)
    return out2.reshape(B, S, D)


# speedup vs baseline: 1.5071x; 1.0841x over previous
import functools

import jax
import jax.numpy as jnp
from jax import lax
from jax.experimental import pallas as pl
from jax.experimental.pallas import tpu as pltpu
from jax.experimental.pallas import tpu_sc as plsc


@functools.lru_cache(maxsize=None)
def _build_sc_kernel(S, T, B, D):
    info = plsc.get_sparse_core_info()
    NC, NS, L = info.num_cores, info.num_subcores, info.num_lanes
    NW = NC * NS
    PQ = L // D
    CW = 128
    BR = CW // D
    NCB = (B * D) // CW
    NSH = NW // NCB
    SQ = S // NSH
    NG = SQ // L
    NP = SQ // PQ
    KR = CW // L

    mesh = plsc.VectorSubcoreMesh(core_axis_name="c", subcore_axis_name="s")

    @functools.partial(
        pl.kernel,
        mesh=mesh,
        out_type=jax.ShapeDtypeStruct((B, S * D), jnp.float32),
        compiler_params=pltpu.CompilerParams(needs_layout_passes=False),
        scratch_types=[
            pltpu.VMEM((T, CW), jnp.float32),
            pltpu.VMEM((T, CW), jnp.float32),
            pltpu.VMEM((SQ,), jnp.float32),
            pltpu.VMEM((SQ,), jnp.int32),
            pltpu.VMEM((SQ,), jnp.float32),
            pltpu.VMEM((1, BR * SQ * D), jnp.float32),
            pltpu.SemaphoreType.DMA,
            pltpu.SemaphoreType.DMA,
            pltpu.SemaphoreType.DMA,
        ],
    )
    def sc_kernel(q_hbm, xt_hbm, sd_hbm, out_hbm,
                  xt2, sdb, q_v, pos_v, w_v, outf,
                  sem_q, sem_sd, sem_xt):
        wid = lax.axis_index("s") * NC + lax.axis_index("c")
        cb = lax.rem(wid, NCB)
        sh = lax.div(wid, NCB)
        col0 = cb * CW
        s0 = sh * SQ

        cp_q = pltpu.async_copy(q_hbm.at[pl.ds(s0, SQ)], q_v, sem_q)
        cp_sd = pltpu.async_copy(sd_hbm.at[:, pl.ds(0, CW)], sdb, sem_sd)
        cp_xt = pltpu.async_copy(xt_hbm.at[:, pl.ds(col0, CW)], xt2, sem_xt)
        cp_q.wait()
        cp_sd.wait()

        lane = lax.iota(jnp.int32, L)
        ones = jnp.ones((L,), jnp.int32)
        zero = jnp.zeros((L,), jnp.int32)

        ts = [sdb[t, pl.ds(0, L)] for t in range(T)]
        spacing = ts[1] - ts[0] + 1e-10

        for g in range(NG):
            q = q_v[pl.ds(g * L, L)]
            acc = zero
            tl = ts[0]
            for t in range(1, T):
                c = ts[t] < q
                acc = acc + jnp.where(c, ones, zero)
                tl = jnp.where(c, ts[t], tl)
            pos_v[pl.ds(g * L, L)] = jnp.minimum(acc, T - 2)
            w_v[pl.ds(g * L, L)] = (q - tl) / spacing

        cp_xt.wait()

        svec = lax.shift_right_logical(lane, 3)
        dvec = lane & (D - 1)

        zvec = jnp.zeros((L,), jnp.int32)

        @plsc.parallel_loop(0, NP, 1, unroll=4)
        def interp(i):
            sv = svec + i * PQ
            pos2 = plsc.load_gather(pos_v, [sv])
            w2 = plsc.load_gather(w_v, [sv])
            flat = pos2 * CW + dvec
            flatb = flat + CW
            inner = sv * D + dvec
            for b in range(BR):
                a = plsc.load_gather(xt2, [zvec, flat + b * D])
                bb = plsc.load_gather(xt2, [zvec, flatb + b * D])
                y = a + w2 * (bb - a)
                plsc.store_scatter(outf, [zvec, inner + b * (SQ * D)], y)

        pltpu.sync_copy(outf.reshape(BR, SQ * D),
                        out_hbm.at[pl.ds(cb * BR, BR),
                                   pl.ds(s0 * D, SQ * D)])

    return sc_kernel


def kernel(query_t, knots, x0, x1, spline_discr):
    (S,) = query_t.shape
    TK, B, D = knots.shape
    T = TK + 2
    sck = _build_sc_kernel(S, T, B, D)
    xt = jnp.concatenate([x0, knots, x1], axis=0).reshape(T, B * D)
    out2 = sck(query_t, xt, spline_discr)
    return out2.reshape(B, S, D)


# speedup vs baseline: 1.5080x; 1.0006x over previous
import functools

import jax
import jax.numpy as jnp
from jax import lax
from jax.experimental import pallas as pl
from jax.experimental.pallas import tpu as pltpu
from jax.experimental.pallas import tpu_sc as plsc


@functools.lru_cache(maxsize=None)
def _build_sc_kernel(S, T, B, D):
    info = plsc.get_sparse_core_info()
    NC, NS, L = info.num_cores, info.num_subcores, info.num_lanes
    NW = NC * NS
    PQ = L // D
    CW = 128
    BR = CW // D
    NCB = (B * D) // CW
    NSH = NW // NCB
    SQ = S // NSH
    NG = SQ // L
    NP = SQ // PQ
    KR = CW // L

    mesh = plsc.VectorSubcoreMesh(core_axis_name="c", subcore_axis_name="s")

    @functools.partial(
        pl.kernel,
        mesh=mesh,
        out_type=jax.ShapeDtypeStruct((B, S * D), jnp.float32),
        compiler_params=pltpu.CompilerParams(needs_layout_passes=False),
        scratch_types=[
            pltpu.VMEM((T, CW), jnp.float32),
            pltpu.VMEM((T, CW), jnp.float32),
            pltpu.VMEM((SQ,), jnp.float32),
            pltpu.VMEM((SQ,), jnp.int32),
            pltpu.VMEM((SQ,), jnp.float32),
            pltpu.VMEM((1, BR * SQ * D), jnp.float32),
            pltpu.SemaphoreType.DMA,
            pltpu.SemaphoreType.DMA,
            pltpu.SemaphoreType.DMA,
        ],
    )
    def sc_kernel(q_hbm, xt_hbm, sd_hbm, out_hbm,
                  xt2, sdb, q_v, pos_v, w_v, outf,
                  sem_q, sem_sd, sem_xt):
        wid = lax.axis_index("s") * NC + lax.axis_index("c")
        cb = lax.rem(wid, NCB)
        sh = lax.div(wid, NCB)
        col0 = cb * CW
        s0 = sh * SQ

        cp_q = pltpu.async_copy(q_hbm.at[pl.ds(s0, SQ)], q_v, sem_q)
        cp_sd = pltpu.async_copy(sd_hbm.at[:, pl.ds(0, CW)], sdb, sem_sd)
        cp_xt = pltpu.async_copy(xt_hbm.at[:, pl.ds(col0, CW)], xt2, sem_xt)
        cp_q.wait()
        cp_sd.wait()

        lane = lax.iota(jnp.int32, L)
        ones = jnp.ones((L,), jnp.int32)
        zero = jnp.zeros((L,), jnp.int32)

        ts = [sdb[t, pl.ds(0, L)] for t in range(T)]
        spacing = ts[1] - ts[0] + 1e-10

        for g in range(NG):
            q = q_v[pl.ds(g * L, L)]
            acc = zero
            tl = ts[0]
            for t in range(1, T):
                c = ts[t] < q
                acc = acc + jnp.where(c, ones, zero)
                tl = jnp.where(c, ts[t], tl)
            pos_v[pl.ds(g * L, L)] = jnp.minimum(acc, T - 2)
            w_v[pl.ds(g * L, L)] = (q - tl) / spacing

        cp_xt.wait()

        svec = lax.shift_right_logical(lane, 3)
        dvec = lane & (D - 1)

        zvec = jnp.zeros((L,), jnp.int32)

        @plsc.parallel_loop(0, NP, 1, unroll=8)
        def interp(i):
            sv = svec + i * PQ
            pos2 = plsc.load_gather(pos_v, [sv])
            w2 = plsc.load_gather(w_v, [sv])
            flat = pos2 * CW + dvec
            flatb = flat + CW
            inner = sv * D + dvec
            for b in range(BR):
                a = plsc.load_gather(xt2, [zvec, flat + b * D])
                bb = plsc.load_gather(xt2, [zvec, flatb + b * D])
                y = a + w2 * (bb - a)
                plsc.store_scatter(outf, [zvec, inner + b * (SQ * D)], y)

        pltpu.sync_copy(outf.reshape(BR, SQ * D),
                        out_hbm.at[pl.ds(cb * BR, BR),
                                   pl.ds(s0 * D, SQ * D)])

    return sc_kernel


def kernel(query_t, knots, x0, x1, spline_discr):
    (S,) = query_t.shape
    TK, B, D = knots.shape
    T = TK + 2
    sck = _build_sc_kernel(S, T, B, D)
    xt = jnp.concatenate([x0, knots, x1], axis=0).reshape(T, B * D)
    out2 = sck(query_t, xt, spline_discr)
    return out2.reshape(B, S, D)
